# Initial kernel scaffold; baseline (speedup 1.0000x reference)
import jax
import jax.numpy as jnp
from jax.experimental import pallas as pl

H, D, K = 4096, 256, 8192
BETA = 0.25
N = 2 * 2048
BM = 256
NBLK = N // BM


def _vq_argmin_kernel(emb_ref, wpre_ref, bpre_ref, cb_ref, e2_ref,
                      z_ref, idx_ref):
    z = jax.lax.dot_general(
        emb_ref[...], wpre_ref[...],
        dimension_numbers=(((1,), (1,)), ((), ())),
        preferred_element_type=jnp.float32)
    z = z + bpre_ref[...]
    z_ref[...] = z
    x2 = jnp.sum(z ** 2, axis=1, keepdims=True)
    xe = jax.lax.dot_general(
        z, cb_ref[...],
        dimension_numbers=(((1,), (1,)), ((), ())),
        preferred_element_type=jnp.float32)
    dist = x2 + e2_ref[...] - 2.0 * xe
    m = jnp.min(dist, axis=1, keepdims=True)
    iota = jax.lax.broadcasted_iota(jnp.int32, dist.shape, 1)
    idx = jnp.min(jnp.where(dist == m, iota, jnp.int32(K)), axis=1)
    idx_ref[0, 0, :] = idx


def _post_kernel(z_ref, xq_ref, wpost_ref, bpost_ref, plog_ref,
                 out_ref, part_ref):
    z = z_ref[...]
    x_q = xq_ref[...]
    x_q_st = z + (x_q - z)
    out = jax.lax.dot_general(
        x_q_st, wpost_ref[...],
        dimension_numbers=(((1,), (1,)), ((), ())),
        preferred_element_type=jnp.float32)
    out_ref[...] = out + bpost_ref[...]
    diff = x_q - z
    sum_sq = jnp.sum(diff * diff)
    sum_plog = jnp.sum(plog_ref[...])
    part = jnp.zeros((1, 128), jnp.float32)
    part = part.at[0, 0].set(sum_sq)
    part = part.at[0, 1].set(sum_plog)
    part_ref[0, ...] = part


def kernel(embed, W_pre, b_pre, codebook, W_post, b_post, prior_logits):
    emb2d = embed.reshape(N, H)
    e2 = jnp.sum(codebook ** 2, axis=1)[None, :]

    z, idx3 = pl.pallas_call(
        _vq_argmin_kernel,
        grid=(NBLK,),
        in_specs=[
            pl.BlockSpec((BM, H), lambda i: (i, 0)),
            pl.BlockSpec((D, H), lambda i: (0, 0)),
            pl.BlockSpec((1, D), lambda i: (0, 0)),
            pl.BlockSpec((K, D), lambda i: (0, 0)),
            pl.BlockSpec((1, K), lambda i: (0, 0)),
        ],
        out_specs=[
            pl.BlockSpec((BM, D), lambda i: (i, 0)),
            pl.BlockSpec((1, 1, BM), lambda i: (i, 0, 0)),
        ],
        out_shape=[
            jax.ShapeDtypeStruct((N, D), jnp.float32),
            jax.ShapeDtypeStruct((NBLK, 1, BM), jnp.int32),
        ],
    )(emb2d, W_pre, b_pre.reshape(1, D), codebook, e2)
    idx = idx3.reshape(N)

    x_q = jnp.take(codebook, idx, axis=0)
    plog = jnp.take(prior_logits, idx).reshape(NBLK, 1, BM)

    embed_hat2d, parts = pl.pallas_call(
        _post_kernel,
        grid=(NBLK,),
        in_specs=[
            pl.BlockSpec((BM, D), lambda i: (i, 0)),
            pl.BlockSpec((BM, D), lambda i: (i, 0)),
            pl.BlockSpec((H, D), lambda i: (0, 0)),
            pl.BlockSpec((1, H), lambda i: (0, 0)),
            pl.BlockSpec((1, 1, BM), lambda i: (i, 0, 0)),
        ],
        out_specs=[
            pl.BlockSpec((BM, H), lambda i: (i, 0)),
            pl.BlockSpec((1, 1, 128), lambda i: (i, 0, 0)),
        ],
        out_shape=[
            jax.ShapeDtypeStruct((N, H), jnp.float32),
            jax.ShapeDtypeStruct((NBLK, 1, 128), jnp.float32),
        ],
    )(z, x_q, W_post, b_post.reshape(1, H), plog)

    embed_hat = embed_hat2d.reshape(embed.shape)
    sum_sq = jnp.sum(parts[:, 0, 0])
    sum_plog = jnp.sum(parts[:, 0, 1])
    mean_sq = sum_sq / (N * D)
    vq_loss = mean_sq + BETA * mean_sq
    lse = jax.nn.logsumexp(prior_logits)
    rate_bits = (N * lse - sum_plog) / jnp.log(2.0)
    return (embed_hat, idx, rate_bits, vq_loss)


# speedup vs baseline: 1.1145x; 1.1145x over previous
import jax
import jax.numpy as jnp
from jax.experimental import pallas as pl

H, D, K = 4096, 256, 8192
BETA = 0.25
N = 2 * 2048
BM = 256
NBLK = N // BM


def _vq_argmin_kernel(emb_ref, wpre_ref, bpre_ref, cb_ref, e2_ref,
                      z_ref, idx_ref):
    z = jax.lax.dot_general(
        emb_ref[...], wpre_ref[...],
        dimension_numbers=(((1,), (1,)), ((), ())),
        preferred_element_type=jnp.float32)
    z = z + bpre_ref[...]
    z_ref[...] = z
    x2 = jnp.sum(z ** 2, axis=1, keepdims=True)
    xe = jax.lax.dot_general(
        z, cb_ref[...],
        dimension_numbers=(((1,), (1,)), ((), ())),
        preferred_element_type=jnp.float32)
    dist = x2 + e2_ref[...] - 2.0 * xe
    m = jnp.min(dist, axis=1, keepdims=True)
    iota = jax.lax.broadcasted_iota(jnp.int32, dist.shape, 1)
    idx = jnp.min(jnp.where(dist == m, iota, jnp.int32(K)), axis=1)
    idx_ref[0, 0, :] = idx


def _post_kernel(z_ref, xq_ref, wpost_ref, bpost_ref, plog_ref,
                 out_ref, part_ref):
    z = z_ref[...]
    x_q = xq_ref[...]
    x_q_st = z + (x_q - z)
    out = jax.lax.dot_general(
        x_q_st, wpost_ref[...],
        dimension_numbers=(((1,), (1,)), ((), ())),
        preferred_element_type=jnp.float32)
    out_ref[...] = out + bpost_ref[...]
    diff = x_q - z
    sum_sq = jnp.sum(diff * diff)
    sum_plog = jnp.sum(plog_ref[...])
    lane = jax.lax.broadcasted_iota(jnp.int32, (1, 128), 1)
    part = jnp.where(lane == 0, sum_sq, jnp.where(lane == 1, sum_plog, 0.0))
    part_ref[0, ...] = part


def kernel(embed, W_pre, b_pre, codebook, W_post, b_post, prior_logits):
    emb2d = embed.reshape(N, H)
    e2 = jnp.sum(codebook ** 2, axis=1)[None, :]

    z, idx3 = pl.pallas_call(
        _vq_argmin_kernel,
        grid=(NBLK,),
        in_specs=[
            pl.BlockSpec((BM, H), lambda i: (i, 0)),
            pl.BlockSpec((D, H), lambda i: (0, 0)),
            pl.BlockSpec((1, D), lambda i: (0, 0)),
            pl.BlockSpec((K, D), lambda i: (0, 0)),
            pl.BlockSpec((1, K), lambda i: (0, 0)),
        ],
        out_specs=[
            pl.BlockSpec((BM, D), lambda i: (i, 0)),
            pl.BlockSpec((1, 1, BM), lambda i: (i, 0, 0)),
        ],
        out_shape=[
            jax.ShapeDtypeStruct((N, D), jnp.float32),
            jax.ShapeDtypeStruct((NBLK, 1, BM), jnp.int32),
        ],
    )(emb2d, W_pre, b_pre.reshape(1, D), codebook, e2)
    idx = idx3.reshape(N)

    x_q = jnp.take(codebook, idx, axis=0)
    plog = jnp.take(prior_logits, idx).reshape(NBLK, 1, BM)

    embed_hat2d, parts = pl.pallas_call(
        _post_kernel,
        grid=(NBLK,),
        in_specs=[
            pl.BlockSpec((BM, D), lambda i: (i, 0)),
            pl.BlockSpec((BM, D), lambda i: (i, 0)),
            pl.BlockSpec((H, D), lambda i: (0, 0)),
            pl.BlockSpec((1, H), lambda i: (0, 0)),
            pl.BlockSpec((1, 1, BM), lambda i: (i, 0, 0)),
        ],
        out_specs=[
            pl.BlockSpec((BM, H), lambda i: (i, 0)),
            pl.BlockSpec((1, 1, 128), lambda i: (i, 0, 0)),
        ],
        out_shape=[
            jax.ShapeDtypeStruct((N, H), jnp.float32),
            jax.ShapeDtypeStruct((NBLK, 1, 128), jnp.float32),
        ],
    )(z, x_q, W_post, b_post.reshape(1, H), plog)

    embed_hat = embed_hat2d.reshape(embed.shape)
    sum_sq = jnp.sum(parts[:, 0, 0])
    sum_plog = jnp.sum(parts[:, 0, 1])
    mean_sq = sum_sq / (N * D)
    vq_loss = mean_sq + BETA * mean_sq
    lse = jax.nn.logsumexp(prior_logits)
    rate_bits = (N * lse - sum_plog) / jnp.log(2.0)
    return (embed_hat, idx, rate_bits, vq_loss)
